# baseline (speedup 1.0000x reference)
import functools
import math
import jax
import jax.numpy as jnp
from jax import lax
from jax.experimental import pallas as pl
from jax.experimental.pallas import tpu as pltpu
from jax.experimental.pallas import tpu_sc as plsc

_K = 327
_ROWS = 128
_COLS = 32768
_NW = 32
_ROWS_PER_W = _ROWS // _NW
_SLICES = _COLS // 16
_INTERP_ITERS = 8
_MAX_ITERS = 50


def _f2k_s(v):
    b = lax.bitcast_convert_type(v, jnp.int32)
    return jnp.where(b < 0, b ^ jnp.int32(0x7FFFFFFF), b)


def _k2f_s(k):
    b = jnp.where(k < 0, k ^ jnp.int32(0x7FFFFFFF), k)
    return lax.bitcast_convert_type(b, jnp.float32)


def _rcp(x):
    y = lax.bitcast_convert_type(
        jnp.int32(0x7EF127EA) - lax.bitcast_convert_type(x, jnp.int32),
        jnp.float32,
    )
    y = y * (jnp.float32(2.0) - x * y)
    y = y * (jnp.float32(2.0) - x * y)
    y = y * (jnp.float32(2.0) - x * y)
    return y


def _log2_approx(c):
    b = lax.bitcast_convert_type(c, jnp.int32)
    return b.astype(jnp.float32) * jnp.float32(1.0 / 8388608.0) - jnp.float32(127.0)


def _sc_kernel(x_hbm, o_hbm, row_v, sem):
    wid = lax.axis_index("s") * 2 + lax.axis_index("c")
    kf = jnp.float32(_K)
    one = jnp.int32(1)
    l2k = jnp.float32(math.log2(float(_K)))

    for rr in range(_ROWS_PER_W):
        r = wid * _ROWS_PER_W + rr
        pltpu.sync_copy(x_hbm.at[r], row_v)

        def count_ge(p):
            def cbody(i, acc):
                v = row_v[pl.ds(i * 16, 16)]
                return acc + jnp.where(v >= p, one, jnp.int32(0))

            acc = lax.fori_loop(0, _SLICES, cbody, jnp.zeros((16,), jnp.int32))
            return jnp.sum(acc).astype(jnp.float32)

        neg_inf_k = _f2k_s(jnp.float32(-jnp.inf))
        pos_nan_k = _f2k_s(jnp.float32(jnp.inf)) + one

        def cond(st):
            j, lo_k, hi_k, llo, lhi, done, bound = st
            return jnp.logical_and(j < _MAX_ITERS, done == 0)

        def body(st):
            j, lo_k, hi_k, llo, lhi, done, bound = st
            lo_v = _k2f_s(lo_k)
            hi_v = _k2f_s(hi_k)

            width1 = hi_k == lo_k + one
            bound = jnp.where(width1, lo_v, bound)
            done = jnp.where(width1, one, done)

            rt = (l2k - lhi) * _rcp(llo - lhi)
            p_interp = hi_v + (lo_v - hi_v) * rt
            pk_i = _f2k_s(p_interp)
            pk_m = (lo_k >> 1) + (hi_k >> 1) + (lo_k & hi_k & one)
            pk = jnp.where(j < _INTERP_ITERS, pk_i, pk_m)
            pk = jnp.clip(pk, lo_k + one, hi_k - one)
            p_v = _k2f_s(pk)

            cnt = count_ge(p_v)

            hit = jnp.logical_and(done == 0, cnt == kf)
            bound = jnp.where(hit, p_v, bound)
            done = jnp.where(hit, one, done)

            lp = _log2_approx(jnp.maximum(cnt, jnp.float32(0.5)))
            live = done == 0
            take_lo = jnp.logical_and(live, cnt >= kf)
            take_hi = jnp.logical_and(live, cnt < kf)
            lo_k = jnp.where(take_lo, pk, lo_k)
            llo = jnp.where(take_lo, lp, llo)
            hi_k = jnp.where(take_hi, pk, hi_k)
            lhi = jnp.where(take_hi, lp, lhi)
            return j + one, lo_k, hi_k, llo, lhi, done, bound

        st0 = (
            jnp.int32(0),
            neg_inf_k,
            pos_nan_k,
            _log2_approx(jnp.float32(_COLS)),
            jnp.float32(-1.0),
            jnp.int32(0),
            jnp.float32(0.0),
        )
        st = lax.while_loop(cond, body, st0)
        _, lo_k, _, _, _, done, bound = st
        bound = jnp.where(done == 0, _k2f_s(lo_k), bound)

        def mbody(i, carry):
            v = row_v[pl.ds(i * 16, 16)]
            row_v[pl.ds(i * 16, 16)] = jnp.where(v >= bound, v, jnp.float32(0.0))
            return carry

        lax.fori_loop(0, _SLICES, mbody, jnp.int32(0))
        pltpu.sync_copy(row_v, o_hbm.at[r])


def kernel(x):
    mesh = plsc.VectorSubcoreMesh(core_axis_name="c", subcore_axis_name="s")
    f = functools.partial(
        pl.kernel,
        mesh=mesh,
        out_type=jax.ShapeDtypeStruct((_ROWS, _COLS), jnp.float32),
        scratch_types=[
            pltpu.VMEM((_COLS,), jnp.float32),
            pltpu.SemaphoreType.DMA,
        ],
        compiler_params=pltpu.CompilerParams(needs_layout_passes=False),
    )(_sc_kernel)
    return f(x)


# speedup vs baseline: 5.0393x; 5.0393x over previous
import functools
import math
import jax
import jax.numpy as jnp
from jax import lax
from jax.experimental import pallas as pl
from jax.experimental.pallas import tpu as pltpu
from jax.experimental.pallas import tpu_sc as plsc

_K = 327
_ROWS = 128
_COLS = 32768
_NW = 32
_ROWS_PER_W = _ROWS // _NW
_SLICES = _COLS // 16
_INTERP_ITERS = 8
_MAX_ITERS = 50


def _f2k_s(v):
    b = lax.bitcast_convert_type(v, jnp.int32)
    return jnp.where(b < 0, b ^ jnp.int32(0x7FFFFFFF), b)


def _k2f_s(k):
    b = jnp.where(k < 0, k ^ jnp.int32(0x7FFFFFFF), k)
    return lax.bitcast_convert_type(b, jnp.float32)


def _rcp(x):
    y = lax.bitcast_convert_type(
        jnp.int32(0x7EF127EA) - lax.bitcast_convert_type(x, jnp.int32),
        jnp.float32,
    )
    y = y * (jnp.float32(2.0) - x * y)
    y = y * (jnp.float32(2.0) - x * y)
    y = y * (jnp.float32(2.0) - x * y)
    return y


def _log2_approx(c):
    b = lax.bitcast_convert_type(c, jnp.int32)
    return b.astype(jnp.float32) * jnp.float32(1.0 / 8388608.0) - jnp.float32(127.0)


def _sc_kernel(x_hbm, o_hbm, row_v, sem):
    wid = lax.axis_index("s") * 2 + lax.axis_index("c")
    kf = jnp.float32(_K)
    one = jnp.int32(1)
    l2k = jnp.float32(math.log2(float(_K)))

    for rr in range(_ROWS_PER_W):
        r = wid * _ROWS_PER_W + rr
        pltpu.sync_copy(x_hbm.at[r], row_v)

        def count_ge(p):
            def cbody(i, acc):
                base = i * 128
                for u in range(8):
                    v = row_v[pl.ds(base + u * 16, 16)]
                    acc = acc + jnp.where(v >= p, one, jnp.int32(0))
                return acc

            acc = lax.fori_loop(
                0, _SLICES // 8, cbody, jnp.zeros((16,), jnp.int32)
            )
            return jnp.sum(acc).astype(jnp.float32)

        neg_inf_k = _f2k_s(jnp.float32(-jnp.inf))
        pos_nan_k = _f2k_s(jnp.float32(jnp.inf)) + one

        def cond(st):
            j, lo_k, hi_k, llo, lhi, done, bound = st
            return jnp.logical_and(j < _MAX_ITERS, done == 0)

        def body(st):
            j, lo_k, hi_k, llo, lhi, done, bound = st
            lo_v = _k2f_s(lo_k)
            hi_v = _k2f_s(hi_k)

            width1 = hi_k == lo_k + one
            bound = jnp.where(width1, lo_v, bound)
            done = jnp.where(width1, one, done)

            rt = (l2k - lhi) * _rcp(llo - lhi)
            p_interp = hi_v + (lo_v - hi_v) * rt
            pk_i = _f2k_s(p_interp)
            pk_m = (lo_k >> 1) + (hi_k >> 1) + (lo_k & hi_k & one)
            pk = jnp.where(j < _INTERP_ITERS, pk_i, pk_m)
            pk = jnp.clip(pk, lo_k + one, hi_k - one)
            p_v = _k2f_s(pk)

            cnt = count_ge(p_v)

            hit = jnp.logical_and(done == 0, cnt == kf)
            bound = jnp.where(hit, p_v, bound)
            done = jnp.where(hit, one, done)

            lp = _log2_approx(jnp.maximum(cnt, jnp.float32(0.5)))
            live = done == 0
            take_lo = jnp.logical_and(live, cnt >= kf)
            take_hi = jnp.logical_and(live, cnt < kf)
            lo_k = jnp.where(take_lo, pk, lo_k)
            llo = jnp.where(take_lo, lp, llo)
            hi_k = jnp.where(take_hi, pk, hi_k)
            lhi = jnp.where(take_hi, lp, lhi)
            return j + one, lo_k, hi_k, llo, lhi, done, bound

        st0 = (
            jnp.int32(0),
            neg_inf_k,
            pos_nan_k,
            _log2_approx(jnp.float32(_COLS)),
            jnp.float32(-1.0),
            jnp.int32(0),
            jnp.float32(0.0),
        )
        st = lax.while_loop(cond, body, st0)
        _, lo_k, _, _, _, done, bound = st
        bound = jnp.where(done == 0, _k2f_s(lo_k), bound)

        def mbody(i, carry):
            base = i * 128
            for u in range(8):
                v = row_v[pl.ds(base + u * 16, 16)]
                row_v[pl.ds(base + u * 16, 16)] = jnp.where(
                    v >= bound, v, jnp.float32(0.0)
                )
            return carry

        lax.fori_loop(0, _SLICES // 8, mbody, jnp.int32(0))
        pltpu.sync_copy(row_v, o_hbm.at[r])


def kernel(x):
    mesh = plsc.VectorSubcoreMesh(core_axis_name="c", subcore_axis_name="s")
    f = functools.partial(
        pl.kernel,
        mesh=mesh,
        out_type=jax.ShapeDtypeStruct((_ROWS, _COLS), jnp.float32),
        scratch_types=[
            pltpu.VMEM((_COLS,), jnp.float32),
            pltpu.SemaphoreType.DMA,
        ],
        compiler_params=pltpu.CompilerParams(needs_layout_passes=False),
    )(_sc_kernel)
    return f(x)


# speedup vs baseline: 15.4630x; 3.0685x over previous
import jax
import jax.numpy as jnp
from jax.experimental import pallas as pl
from jax.experimental.pallas import tpu as pltpu

_K = 327
_ROWS = 128
_COLS = 32768
_BLOCK_R = 32
_STAT_COLS = 2048
_INTERP_ITERS = 8
_MAX_ITERS = 50


def _f2k(v):
    b = jax.lax.bitcast_convert_type(v, jnp.int32)
    return jnp.where(b < 0, b ^ jnp.int32(0x7FFFFFFF), b)


def _k2f(k):
    b = jnp.where(k < 0, k ^ jnp.int32(0x7FFFFFFF), k)
    return jax.lax.bitcast_convert_type(b, jnp.float32)


def _topk_mask_kernel(x_ref, o_ref):
    x = x_ref[...]
    nrows = x.shape[0]
    ncols = x.shape[1]
    one = jnp.int32(1)
    kf = jnp.float32(_K)

    xs = x[:, :_STAT_COLS]
    s1 = jnp.sum(xs, axis=1, keepdims=True)
    s2 = jnp.sum(xs * xs, axis=1, keepdims=True)
    mu = s1 / _STAT_COLS
    sd = jnp.sqrt(jnp.maximum(s2 / _STAT_COLS - mu * mu, 0.0))
    a_v = mu + 1.65 * sd
    b_v = mu + 3.8 * sd

    cnt_a = jnp.sum(jnp.where(x >= a_v, 1.0, 0.0), axis=1, keepdims=True)
    cnt_b = jnp.sum(jnp.where(x >= b_v, 1.0, 0.0), axis=1, keepdims=True)

    neg_inf_k = _f2k(jnp.float32(-jnp.inf))
    pos_nan_k = _f2k(jnp.float32(jnp.inf)) + one
    lo_ok = jnp.logical_and(cnt_a >= kf, a_v == a_v)
    lo_k = jnp.where(lo_ok, _f2k(a_v), neg_inf_k)
    cnt_lo = jnp.where(lo_ok, cnt_a, jnp.float32(ncols))
    hi_ok = jnp.logical_and(cnt_b < kf, b_v == b_v)
    hi_k = jnp.where(hi_ok, _f2k(b_v), pos_nan_k)
    cnt_hi = jnp.where(hi_ok, cnt_b, jnp.float32(0.0))

    logk = jnp.float32(jnp.log(float(_K)))
    llo = jnp.log(cnt_lo)
    lhi = jnp.log(jnp.maximum(cnt_hi, 0.5))

    def cond(state):
        j = state[0]
        done = state[5]
        return jnp.logical_and(j < _MAX_ITERS, jnp.sum(done) < nrows)

    def body(state):
        j, lo_k, hi_k, llo, lhi, done, bound = state
        lo_v = _k2f(lo_k)
        hi_v = _k2f(hi_k)

        width1 = jnp.where(hi_k == lo_k + one, 1 - done, 0)
        bound = jnp.where(width1 == 1, lo_v, bound)
        done = done | width1

        r = (logk - lhi) / (llo - lhi)
        p_interp = hi_v + (lo_v - hi_v) * r
        pk_i = _f2k(p_interp)
        pk_m = (lo_k >> 1) + (hi_k >> 1) + (lo_k & hi_k & one)
        pk = jnp.where(j < _INTERP_ITERS, pk_i, pk_m)
        pk = jnp.clip(pk, lo_k + one, hi_k - one)
        p_v = _k2f(pk)

        cnt = jnp.sum(jnp.where(x >= p_v, 1.0, 0.0), axis=1, keepdims=True)

        hit_k = jnp.where(cnt == kf, 1 - done, 0)
        bound = jnp.where(hit_k == 1, p_v, bound)
        done = done | hit_k

        lp = jnp.log(jnp.maximum(cnt, 0.5))
        live = done == 0
        take_lo = jnp.logical_and(live, cnt >= kf)
        take_hi = jnp.logical_and(live, cnt < kf)
        lo_k = jnp.where(take_lo, pk, lo_k)
        llo = jnp.where(take_lo, lp, llo)
        hi_k = jnp.where(take_hi, pk, hi_k)
        lhi = jnp.where(take_hi, lp, lhi)
        return j + one, lo_k, hi_k, llo, lhi, done, bound

    state0 = (
        jnp.int32(0),
        lo_k,
        hi_k,
        llo,
        lhi,
        jnp.zeros((nrows, 1), jnp.int32),
        jnp.zeros((nrows, 1), jnp.float32),
    )
    st = jax.lax.while_loop(cond, body, state0)
    _, lo_k, _, _, _, done, bound = st
    bound = jnp.where(done == 0, _k2f(lo_k), bound)

    o_ref[...] = jnp.where(x >= bound, x, jnp.float32(0.0))


def kernel(x):
    return pl.pallas_call(
        _topk_mask_kernel,
        grid=(_ROWS // _BLOCK_R,),
        in_specs=[pl.BlockSpec((_BLOCK_R, _COLS), lambda r: (r, 0))],
        out_specs=pl.BlockSpec((_BLOCK_R, _COLS), lambda r: (r, 0)),
        out_shape=jax.ShapeDtypeStruct((_ROWS, _COLS), jnp.float32),
    )(x)


# speedup vs baseline: 15.7382x; 1.0178x over previous
import jax
import jax.numpy as jnp
from jax.experimental import pallas as pl
from jax.experimental.pallas import tpu as pltpu

_K = 327
_ROWS = 128
_COLS = 32768
_BLOCK_R = 64
_STAT_COLS = 2048
_INTERP_ITERS = 8
_MAX_ITERS = 50


def _f2k(v):
    b = jax.lax.bitcast_convert_type(v, jnp.int32)
    return jnp.where(b < 0, b ^ jnp.int32(0x7FFFFFFF), b)


def _k2f(k):
    b = jnp.where(k < 0, k ^ jnp.int32(0x7FFFFFFF), k)
    return jax.lax.bitcast_convert_type(b, jnp.float32)


def _topk_mask_kernel(x_ref, o_ref):
    x = x_ref[...]
    nrows = x.shape[0]
    ncols = x.shape[1]
    one = jnp.int32(1)
    kf = jnp.float32(_K)

    xs = x[:, :_STAT_COLS]
    s1 = jnp.sum(xs, axis=1, keepdims=True)
    s2 = jnp.sum(xs * xs, axis=1, keepdims=True)
    mu = s1 / _STAT_COLS
    sd = jnp.sqrt(jnp.maximum(s2 / _STAT_COLS - mu * mu, 0.0))
    a_v = mu + 1.65 * sd
    b_v = mu + 3.8 * sd

    cnt_a = jnp.sum(jnp.where(x >= a_v, 1.0, 0.0), axis=1, keepdims=True)
    cnt_b = jnp.sum(jnp.where(x >= b_v, 1.0, 0.0), axis=1, keepdims=True)

    neg_inf_k = _f2k(jnp.float32(-jnp.inf))
    pos_nan_k = _f2k(jnp.float32(jnp.inf)) + one
    lo_ok = jnp.logical_and(cnt_a >= kf, a_v == a_v)
    lo_k = jnp.where(lo_ok, _f2k(a_v), neg_inf_k)
    cnt_lo = jnp.where(lo_ok, cnt_a, jnp.float32(ncols))
    hi_ok = jnp.logical_and(cnt_b < kf, b_v == b_v)
    hi_k = jnp.where(hi_ok, _f2k(b_v), pos_nan_k)
    cnt_hi = jnp.where(hi_ok, cnt_b, jnp.float32(0.0))

    logk = jnp.float32(jnp.log(float(_K)))
    llo = jnp.log(cnt_lo)
    lhi = jnp.log(jnp.maximum(cnt_hi, 0.5))

    def cond(state):
        j = state[0]
        done = state[5]
        return jnp.logical_and(j < _MAX_ITERS, jnp.sum(done) < nrows)

    def body(state):
        j, lo_k, hi_k, llo, lhi, done, bound = state
        lo_v = _k2f(lo_k)
        hi_v = _k2f(hi_k)

        width1 = jnp.where(hi_k == lo_k + one, 1 - done, 0)
        bound = jnp.where(width1 == 1, lo_v, bound)
        done = done | width1

        r = (logk - lhi) / (llo - lhi)
        p_interp = hi_v + (lo_v - hi_v) * r
        pk_i = _f2k(p_interp)
        pk_m = (lo_k >> 1) + (hi_k >> 1) + (lo_k & hi_k & one)
        pk = jnp.where(j < _INTERP_ITERS, pk_i, pk_m)
        pk = jnp.clip(pk, lo_k + one, hi_k - one)
        p_v = _k2f(pk)

        cnt = jnp.sum(jnp.where(x >= p_v, 1.0, 0.0), axis=1, keepdims=True)

        hit_k = jnp.where(cnt == kf, 1 - done, 0)
        bound = jnp.where(hit_k == 1, p_v, bound)
        done = done | hit_k

        lp = jnp.log(jnp.maximum(cnt, 0.5))
        live = done == 0
        take_lo = jnp.logical_and(live, cnt >= kf)
        take_hi = jnp.logical_and(live, cnt < kf)
        lo_k = jnp.where(take_lo, pk, lo_k)
        llo = jnp.where(take_lo, lp, llo)
        hi_k = jnp.where(take_hi, pk, hi_k)
        lhi = jnp.where(take_hi, lp, lhi)
        return j + one, lo_k, hi_k, llo, lhi, done, bound

    state0 = (
        jnp.int32(0),
        lo_k,
        hi_k,
        llo,
        lhi,
        jnp.zeros((nrows, 1), jnp.int32),
        jnp.zeros((nrows, 1), jnp.float32),
    )
    st = jax.lax.while_loop(cond, body, state0)
    _, lo_k, _, _, _, done, bound = st
    bound = jnp.where(done == 0, _k2f(lo_k), bound)

    o_ref[...] = jnp.where(x >= bound, x, jnp.float32(0.0))


def kernel(x):
    return pl.pallas_call(
        _topk_mask_kernel,
        grid=(_ROWS // _BLOCK_R,),
        in_specs=[pl.BlockSpec((_BLOCK_R, _COLS), lambda r: (r, 0))],
        out_specs=pl.BlockSpec((_BLOCK_R, _COLS), lambda r: (r, 0)),
        out_shape=jax.ShapeDtypeStruct((_ROWS, _COLS), jnp.float32),
    )(x)


# speedup vs baseline: 15.7545x; 1.0010x over previous
import jax
import jax.numpy as jnp
from jax.experimental import pallas as pl
from jax.experimental.pallas import tpu as pltpu

_K = 327
_ROWS = 128
_COLS = 32768
_BLOCK_R = 128
_STAT_COLS = 2048
_INTERP_ITERS = 8
_MAX_ITERS = 50


def _f2k(v):
    b = jax.lax.bitcast_convert_type(v, jnp.int32)
    return jnp.where(b < 0, b ^ jnp.int32(0x7FFFFFFF), b)


def _k2f(k):
    b = jnp.where(k < 0, k ^ jnp.int32(0x7FFFFFFF), k)
    return jax.lax.bitcast_convert_type(b, jnp.float32)


def _topk_mask_kernel(x_ref, o_ref):
    x = x_ref[...]
    nrows = x.shape[0]
    ncols = x.shape[1]
    one = jnp.int32(1)
    kf = jnp.float32(_K)

    xs = x[:, :_STAT_COLS]
    s1 = jnp.sum(xs, axis=1, keepdims=True)
    s2 = jnp.sum(xs * xs, axis=1, keepdims=True)
    mu = s1 / _STAT_COLS
    sd = jnp.sqrt(jnp.maximum(s2 / _STAT_COLS - mu * mu, 0.0))
    a_v = mu + 1.65 * sd
    b_v = mu + 3.8 * sd

    cnt_a = jnp.sum(jnp.where(x >= a_v, 1.0, 0.0), axis=1, keepdims=True)
    cnt_b = jnp.sum(jnp.where(x >= b_v, 1.0, 0.0), axis=1, keepdims=True)

    neg_inf_k = _f2k(jnp.float32(-jnp.inf))
    pos_nan_k = _f2k(jnp.float32(jnp.inf)) + one
    lo_ok = jnp.logical_and(cnt_a >= kf, a_v == a_v)
    lo_k = jnp.where(lo_ok, _f2k(a_v), neg_inf_k)
    cnt_lo = jnp.where(lo_ok, cnt_a, jnp.float32(ncols))
    hi_ok = jnp.logical_and(cnt_b < kf, b_v == b_v)
    hi_k = jnp.where(hi_ok, _f2k(b_v), pos_nan_k)
    cnt_hi = jnp.where(hi_ok, cnt_b, jnp.float32(0.0))

    logk = jnp.float32(jnp.log(float(_K)))
    llo = jnp.log(cnt_lo)
    lhi = jnp.log(jnp.maximum(cnt_hi, 0.5))

    def cond(state):
        j = state[0]
        done = state[5]
        return jnp.logical_and(j < _MAX_ITERS, jnp.sum(done) < nrows)

    def body(state):
        j, lo_k, hi_k, llo, lhi, done, bound = state
        lo_v = _k2f(lo_k)
        hi_v = _k2f(hi_k)

        width1 = jnp.where(hi_k == lo_k + one, 1 - done, 0)
        bound = jnp.where(width1 == 1, lo_v, bound)
        done = done | width1

        r = (logk - lhi) / (llo - lhi)
        p_interp = hi_v + (lo_v - hi_v) * r
        pk_i = _f2k(p_interp)
        pk_m = (lo_k >> 1) + (hi_k >> 1) + (lo_k & hi_k & one)
        pk = jnp.where(j < _INTERP_ITERS, pk_i, pk_m)
        pk = jnp.clip(pk, lo_k + one, hi_k - one)
        p_v = _k2f(pk)

        cnt = jnp.sum(jnp.where(x >= p_v, 1.0, 0.0), axis=1, keepdims=True)

        hit_k = jnp.where(cnt == kf, 1 - done, 0)
        bound = jnp.where(hit_k == 1, p_v, bound)
        done = done | hit_k

        lp = jnp.log(jnp.maximum(cnt, 0.5))
        live = done == 0
        take_lo = jnp.logical_and(live, cnt >= kf)
        take_hi = jnp.logical_and(live, cnt < kf)
        lo_k = jnp.where(take_lo, pk, lo_k)
        llo = jnp.where(take_lo, lp, llo)
        hi_k = jnp.where(take_hi, pk, hi_k)
        lhi = jnp.where(take_hi, lp, lhi)
        return j + one, lo_k, hi_k, llo, lhi, done, bound

    state0 = (
        jnp.int32(0),
        lo_k,
        hi_k,
        llo,
        lhi,
        jnp.zeros((nrows, 1), jnp.int32),
        jnp.zeros((nrows, 1), jnp.float32),
    )
    st = jax.lax.while_loop(cond, body, state0)
    _, lo_k, _, _, _, done, bound = st
    bound = jnp.where(done == 0, _k2f(lo_k), bound)

    o_ref[...] = jnp.where(x >= bound, x, jnp.float32(0.0))


def kernel(x):
    return pl.pallas_call(
        _topk_mask_kernel,
        grid=(_ROWS // _BLOCK_R,),
        in_specs=[pl.BlockSpec((_BLOCK_R, _COLS), lambda r: (r, 0))],
        out_specs=pl.BlockSpec((_BLOCK_R, _COLS), lambda r: (r, 0)),
        out_shape=jax.ShapeDtypeStruct((_ROWS, _COLS), jnp.float32),
    )(x)
